# Initial kernel scaffold; baseline (speedup 1.0000x reference)
import jax
import jax.numpy as jnp
from jax.experimental import pallas as pl


def kernel(x, A, weight, bias):
    raise NotImplementedError("write your pallas kernel here")


# speedup vs baseline: 2.2397x; 2.2397x over previous
import functools

import jax
import jax.numpy as jnp
from jax.experimental import pallas as pl
from jax.experimental.pallas import tpu as pltpu


def _gcn_kernel(x_ref, abd_ref, w_ref, b_ref, o_ref, *, n_sub, sub):
    x = x_ref[0]
    w = w_ref[...]
    b = b_ref[...]
    y = jnp.dot(w, x.astype(jnp.bfloat16), preferred_element_type=jnp.float32)
    y = (y + b).astype(jnp.bfloat16)
    abd = abd_ref[0]
    for s in range(n_sub):
        sl = pl.ds(s * sub, sub)
        z = jnp.dot(y[:, s * sub:(s + 1) * sub], abd,
                    preferred_element_type=jnp.float32)
        o_ref[0, :, sl] = z


def _graph_conv(x, A, weight, bias, split):
    n, c_in, t, v = x.shape
    c_out = weight.shape[0]

    tile_t = max(1, 256 // v) if (256 % v == 0 and t % max(1, 256 // v) == 0) else 1
    sub = tile_t * v
    lanes = t * v
    while lanes % (split * sub) != 0:
        split //= 2
    blk_l = lanes // split
    n_sub = blk_l // sub

    x_flat = x.reshape(n, c_in, lanes)
    eye = jnp.eye(tile_t, dtype=A.dtype)
    a_bd = jnp.einsum("ij,nvw->nivjw", eye, A).reshape(n, sub, sub)
    a_bd = a_bd.astype(jnp.bfloat16)
    w_bf = weight.astype(jnp.bfloat16)
    b2 = bias.reshape(c_out, 1)

    body = functools.partial(_gcn_kernel, n_sub=n_sub, sub=sub)
    out_flat = pl.pallas_call(
        body,
        out_shape=jax.ShapeDtypeStruct((n, c_out, lanes), x.dtype),
        grid=(n, split),
        in_specs=[
            pl.BlockSpec((1, c_in, blk_l), lambda i, j: (i, 0, j)),
            pl.BlockSpec((1, sub, sub), lambda i, j: (i, 0, 0)),
            pl.BlockSpec((c_out, c_in), lambda i, j: (0, 0)),
            pl.BlockSpec((c_out, 1), lambda i, j: (0, 0)),
        ],
        out_specs=pl.BlockSpec((1, c_out, blk_l), lambda i, j: (i, 0, j)),
        compiler_params=pltpu.CompilerParams(
            dimension_semantics=("parallel", "parallel"),
            vmem_limit_bytes=64 * 1024 * 1024,
        ),
    )(x_flat, a_bd, w_bf, b2)

    return out_flat.reshape(n, c_out, t, v)


def kernel(x, A, weight, bias):
    out = _graph_conv(x, A, weight, bias, split=4)
    return out, A


# speedup vs baseline: 4.1320x; 1.8449x over previous
import functools

import jax
import jax.numpy as jnp
from jax.experimental import pallas as pl
from jax.experimental.pallas import tpu as pltpu


def _gcn_kernel(x_ref, abd_ref, w_ref, b_ref, o_ref, *, tile_t, pair_t, v):
    sub = pair_t * v
    w = w_ref[...]
    b = b_ref[...]
    xcat = jnp.concatenate(
        [x_ref[0, :, tt, :] for tt in range(tile_t)], axis=1
    ).astype(jnp.bfloat16)
    y = jnp.dot(w, xcat, preferred_element_type=jnp.float32)
    y = (y + b).astype(jnp.bfloat16)
    abd = abd_ref[0]
    for s in range(tile_t // pair_t):
        z = jnp.dot(y[:, s * sub:(s + 1) * sub], abd,
                    preferred_element_type=jnp.float32)
        for p in range(pair_t):
            o_ref[0, :, s * pair_t + p, :] = z[:, p * v:(p + 1) * v]


def _graph_conv(x, A, weight, bias, tile_t):
    n, c_in, t, v = x.shape
    c_out = weight.shape[0]

    pair_t = max(1, 256 // v) if (256 % v == 0 and t % max(1, 256 // v) == 0) else 1
    while t % tile_t != 0 or tile_t % pair_t != 0:
        tile_t //= 2
    sub = pair_t * v

    eye = jnp.eye(pair_t, dtype=A.dtype)
    a_bd = jnp.einsum("ij,nvw->nivjw", eye, A).reshape(n, sub, sub)
    a_bd = a_bd.astype(jnp.bfloat16)
    w_bf = weight.astype(jnp.bfloat16)
    b2 = bias.reshape(c_out, 1)

    body = functools.partial(_gcn_kernel, tile_t=tile_t, pair_t=pair_t, v=v)
    out = pl.pallas_call(
        body,
        out_shape=jax.ShapeDtypeStruct((n, c_out, t, v), x.dtype),
        grid=(n, t // tile_t),
        in_specs=[
            pl.BlockSpec((1, c_in, tile_t, v), lambda i, j: (i, 0, j, 0)),
            pl.BlockSpec((1, sub, sub), lambda i, j: (i, 0, 0)),
            pl.BlockSpec((c_out, c_in), lambda i, j: (0, 0)),
            pl.BlockSpec((c_out, 1), lambda i, j: (0, 0)),
        ],
        out_specs=pl.BlockSpec((1, c_out, tile_t, v), lambda i, j: (i, 0, j, 0)),
        compiler_params=pltpu.CompilerParams(
            dimension_semantics=("parallel", "parallel"),
            vmem_limit_bytes=64 * 1024 * 1024,
        ),
    )(x, a_bd, w_bf, b2)
    return out


def kernel(x, A, weight, bias):
    out = _graph_conv(x, A, weight, bias, tile_t=16)
    return out, A


# speedup vs baseline: 6.9318x; 1.6776x over previous
import functools

import jax
import jax.numpy as jnp
from jax.experimental import pallas as pl
from jax.experimental.pallas import tpu as pltpu


def _gcn_kernel(x_ref, abd_ref, w_ref, b_ref, o_ref, *, tile_t, pair_t, v):
    sub = pair_t * v
    w = w_ref[...]
    b = b_ref[...]
    c_in = x_ref.shape[1]
    xcat = x_ref[0].reshape(c_in, tile_t * v).astype(jnp.bfloat16)
    y = jnp.dot(w, xcat, preferred_element_type=jnp.float32)
    y = (y + b).astype(jnp.bfloat16)
    abd = abd_ref[0]
    zs = [
        jnp.dot(y[:, s * sub:(s + 1) * sub], abd,
                preferred_element_type=jnp.float32)
        for s in range(tile_t // pair_t)
    ]
    zfull = jnp.concatenate(zs, axis=1)
    o_ref[0] = zfull.reshape(w_ref.shape[0], tile_t, v)


def _graph_conv(x, A, weight, bias, tile_t):
    n, c_in, t, v = x.shape
    c_out = weight.shape[0]

    pair_t = max(1, 256 // v) if (256 % v == 0 and t % max(1, 256 // v) == 0) else 1
    while t % tile_t != 0 or tile_t % pair_t != 0:
        tile_t //= 2
    sub = pair_t * v

    eye = jnp.eye(pair_t, dtype=A.dtype)
    a_bd = jnp.einsum("ij,nvw->nivjw", eye, A).reshape(n, sub, sub)
    a_bd = a_bd.astype(jnp.bfloat16)
    w_bf = weight.astype(jnp.bfloat16)
    b2 = bias.reshape(c_out, 1)

    body = functools.partial(_gcn_kernel, tile_t=tile_t, pair_t=pair_t, v=v)
    out = pl.pallas_call(
        body,
        out_shape=jax.ShapeDtypeStruct((n, c_out, t, v), x.dtype),
        grid=(n, t // tile_t),
        in_specs=[
            pl.BlockSpec((1, c_in, tile_t, v), lambda i, j: (i, 0, j, 0)),
            pl.BlockSpec((1, sub, sub), lambda i, j: (i, 0, 0)),
            pl.BlockSpec((c_out, c_in), lambda i, j: (0, 0)),
            pl.BlockSpec((c_out, 1), lambda i, j: (0, 0)),
        ],
        out_specs=pl.BlockSpec((1, c_out, tile_t, v), lambda i, j: (i, 0, j, 0)),
        compiler_params=pltpu.CompilerParams(
            dimension_semantics=("parallel", "parallel"),
            vmem_limit_bytes=64 * 1024 * 1024,
        ),
    )(x, a_bd, w_bf, b2)
    return out


def kernel(x, A, weight, bias):
    out = _graph_conv(x, A, weight, bias, tile_t=64)
    return out, A


# speedup vs baseline: 7.9804x; 1.1513x over previous
import functools

import jax
import jax.numpy as jnp
from jax.experimental import pallas as pl
from jax.experimental.pallas import tpu as pltpu


def _gcn_kernel(x_ref, a_ref, w_ref, b_ref, o_ref, *, tile_t, pair_t, v):
    sub = pair_t * v
    c_in = x_ref.shape[1]
    c_out = w_ref.shape[0]

    w = w_ref[...].astype(jnp.bfloat16)
    b = jnp.transpose(b_ref[...])

    a = a_ref[0].astype(jnp.bfloat16)
    zv = jnp.zeros((v, v), dtype=jnp.bfloat16)
    rows = []
    for p in range(pair_t):
        rows.append(jnp.concatenate(
            [a if q == p else zv for q in range(pair_t)], axis=1))
    abd = jnp.concatenate(rows, axis=0)

    xcat = x_ref[0].reshape(c_in, tile_t * v).astype(jnp.bfloat16)
    y = jnp.dot(w, xcat, preferred_element_type=jnp.float32)
    y = (y + b).astype(jnp.bfloat16)
    zs = [
        jnp.dot(y[:, s * sub:(s + 1) * sub], abd,
                preferred_element_type=jnp.float32)
        for s in range(tile_t // pair_t)
    ]
    zfull = jnp.concatenate(zs, axis=1)
    o_ref[0] = zfull.reshape(c_out, tile_t, v)


def _graph_conv(x, A, weight, bias, tile_t):
    n, c_in, t, v = x.shape
    c_out = weight.shape[0]

    pair_t = max(1, 256 // v) if (256 % v == 0 and t % max(1, 256 // v) == 0) else 1
    while t % tile_t != 0 or tile_t % pair_t != 0:
        tile_t //= 2

    b2 = bias.reshape(1, c_out)

    body = functools.partial(_gcn_kernel, tile_t=tile_t, pair_t=pair_t, v=v)
    out = pl.pallas_call(
        body,
        out_shape=jax.ShapeDtypeStruct((n, c_out, t, v), x.dtype),
        grid=(n, t // tile_t),
        in_specs=[
            pl.BlockSpec((1, c_in, tile_t, v), lambda i, j: (i, 0, j, 0)),
            pl.BlockSpec((1, v, v), lambda i, j: (i, 0, 0)),
            pl.BlockSpec((c_out, c_in), lambda i, j: (0, 0)),
            pl.BlockSpec((1, c_out), lambda i, j: (0, 0)),
        ],
        out_specs=pl.BlockSpec((1, c_out, tile_t, v), lambda i, j: (i, 0, j, 0)),
        compiler_params=pltpu.CompilerParams(
            dimension_semantics=("parallel", "parallel"),
            vmem_limit_bytes=64 * 1024 * 1024,
        ),
    )(x, A, weight, b2)
    return out


def kernel(x, A, weight, bias):
    out = _graph_conv(x, A, weight, bias, tile_t=64)
    return out, A
